# Initial kernel scaffold; baseline (speedup 1.0000x reference)
import jax
import jax.numpy as jnp
from jax.experimental import pallas as pl


def kernel(x, edge_index, edge_attr, We1, be1, We2, be2, Wn1, bn1, Wn2, bn2):
    raise NotImplementedError("write your pallas kernel here")


# speedup vs baseline: 2.4376x; 2.4376x over previous
import functools

import jax
import jax.numpy as jnp
from jax import lax
from jax.experimental import pallas as pl
from jax.experimental.pallas import tpu as pltpu
from jax.experimental.pallas import tpu_sc as plsc

N = 10000
E = 320000
D = 128
DE = 16
H = 128

NC = 2
NS = 16
NW = NC * NS

SPB = 128
TS = E // SPB
ST_BASE = TS // NW
ST_REM = TS % NW

NP = 10240
ROWS_PER_TILE = NP // NS


import numpy as _np
_MF_NP = _np.repeat(_np.eye(16, dtype=_np.float32), 16, axis=1)


def _silu(v):
    return v * (1.0 / (1.0 + jnp.exp(-v)))


def _proj_body(x_ref, w_ref, b_ref, xr_ref, xc_ref):
    xb = x_ref[...]
    wr = w_ref[DE:DE + D, :]
    wc = w_ref[DE + D:DE + 2 * D, :]
    xr_ref[...] = jnp.dot(xb, wr, preferred_element_type=jnp.float32)
    xc_ref[...] = jnp.dot(xb, wc, preferred_element_type=jnp.float32) + b_ref[...]


def _project(x, We1, be1):
    blk = 1000
    grid = N // blk
    return pl.pallas_call(
        _proj_body,
        grid=(grid,),
        in_specs=[
            pl.BlockSpec((blk, D), lambda i: (i, 0)),
            pl.BlockSpec((DE + 2 * D, H), lambda i: (0, 0)),
            pl.BlockSpec((1, H), lambda i: (0, 0)),
        ],
        out_specs=[
            pl.BlockSpec((blk, H), lambda i: (i, 0)),
            pl.BlockSpec((blk, H), lambda i: (i, 0)),
        ],
        out_shape=[
            jax.ShapeDtypeStruct((N, H), jnp.float32),
            jax.ShapeDtypeStruct((N, H), jnp.float32),
        ],
    )(x, We1, be1.reshape(1, H))


def _sc_gather_body(xr_hbm, xc_hbm, row_hbm, col_hbm, g_hbm,
                    ridx_v, cidx_v, bufr_v, bufc_v, sem0, sem1):
    cid = lax.axis_index("c")
    sid = lax.axis_index("s")
    wid = sid * NC + cid
    n_st = ST_BASE + jnp.where(wid < ST_REM, 1, 0)
    start = wid * ST_BASE + jnp.minimum(wid, ST_REM)

    def stream_body(t, carry):
        st = start + t
        pltpu.sync_copy(row_hbm.at[st], ridx_v)
        pltpu.sync_copy(col_hbm.at[st], cidx_v)
        a = pltpu.async_copy(xr_hbm.at[ridx_v], bufr_v, sem0)
        b = pltpu.async_copy(xc_hbm.at[cidx_v], bufc_v, sem1)
        a.wait()
        b.wait()

        @plsc.parallel_loop(0, SPB, unroll=2)
        def _add(i):
            for k in range(H // 16):
                sl = pl.ds(k * 16, 16)
                bufr_v[i, sl] = bufr_v[i, sl] + bufc_v[i, sl]

        pltpu.sync_copy(bufr_v, g_hbm.at[pl.ds(st * SPB, SPB)])
        return carry

    lax.fori_loop(0, n_st, stream_body, 0)


def _sc_gather(Xr, Xc, row2d, col2d):
    mesh = plsc.VectorSubcoreMesh(core_axis_name="c", subcore_axis_name="s")
    return pl.kernel(
        _sc_gather_body,
        out_type=jax.ShapeDtypeStruct((E, H), jnp.float32),
        mesh=mesh,
        scratch_types=[
            pltpu.VMEM((SPB,), jnp.int32),
            pltpu.VMEM((SPB,), jnp.int32),
            pltpu.VMEM((SPB, H), jnp.float32),
            pltpu.VMEM((SPB, H), jnp.float32),
            pltpu.SemaphoreType.DMA,
            pltpu.SemaphoreType.DMA,
        ],
    )(Xr, Xc, row2d, col2d)


def _edge_body(g_ref, ea_ref, wa_ref, w2_ref, b2_ref, out_ref):
    ea = ea_ref[...]
    h1 = _silu(g_ref[...] +
               jnp.dot(ea, wa_ref[...], preferred_element_type=jnp.float32))
    h2 = _silu(jnp.dot(h1, w2_ref[...], preferred_element_type=jnp.float32)
               + b2_ref[...])
    out_ref[...] = h2 + ea


def _edge_mlp(G, edge_attr, We1, We2, be2):
    blk = 1280
    grid = E // blk
    return pl.pallas_call(
        _edge_body,
        grid=(grid,),
        in_specs=[
            pl.BlockSpec((blk, H), lambda i: (i, 0)),
            pl.BlockSpec((blk, DE), lambda i: (i, 0)),
            pl.BlockSpec((DE, H), lambda i: (0, 0)),
            pl.BlockSpec((H, DE), lambda i: (0, 0)),
            pl.BlockSpec((1, DE), lambda i: (0, 0)),
        ],
        out_specs=pl.BlockSpec((blk, DE), lambda i: (i, 0)),
        out_shape=jax.ShapeDtypeStruct((E, DE), jnp.float32),
    )(G, edge_attr, We1[:DE, :], We2, be2.reshape(1, DE))


NH = NP // 2
NPK = NP // 16


def _sc_scatter_body(eo_hbm, col_hbm, psum_hbm, pcnt_hbm,
                     colv_v, val_v, acc_v, cnt_v):
    cid = lax.axis_index("c")
    sid = lax.axis_index("s")
    wid = sid * NC + cid
    n_st = ST_BASE + jnp.where(wid < ST_REM, 1, 0)
    start = wid * ST_BASE + jnp.minimum(wid, ST_REM)
    lane = lax.iota(jnp.int32, 16)

    @plsc.parallel_loop(0, NPK // 8, unroll=4)
    def _zc(i):
        for k in range(8):
            cnt_v[i, pl.ds(k * 16, 16)] = jnp.zeros((16,), jnp.float32)

    def half_pass(lo, with_counts):
        @plsc.parallel_loop(0, NH // 8, unroll=2)
        def _za(i):
            for k in range(8):
                acc_v[i, pl.ds(k * 16, 16)] = jnp.zeros((16,), jnp.float32)

        def stream_body(t, carry):
            st = start + t
            pltpu.sync_copy(col_hbm.at[st], colv_v)
            pltpu.sync_copy(eo_hbm.at[pl.ds(st * (SPB // 8), SPB // 8)], val_v)

            def gbody(g, carry2):
                cv = colv_v[pl.ds(g * 16, 16)]
                for j in range(16):
                    c = cv[j]
                    rel = c - lo

                    @pl.when((rel >= 0) & (rel < NH))
                    def _(c=c, rel=rel, j=j):
                        row = rel // 8
                        off = (rel - row * 8) * 16
                        ev = val_v[g * 2 + j // 8, pl.ds((j % 8) * 16, 16)]
                        acc_v[row, pl.ds(off, 16)] = (
                            acc_v[row, pl.ds(off, 16)] + ev)

                    if with_counts:
                        rowc = c // 128
                        offc = ((c // 16) % 8) * 16
                        m = c % 16
                        cnt_v[rowc, pl.ds(offc, 16)] = (
                            cnt_v[rowc, pl.ds(offc, 16)]
                            + jnp.where(lane == m, 1.0, 0.0).astype(jnp.float32))
                return carry2

            lax.fori_loop(0, SPB // 16, gbody, 0)
            return carry

        lax.fori_loop(0, n_st, stream_body, 0)
        off8 = pl.multiple_of((wid * NP + lo) // 8, 8)
        pltpu.sync_copy(acc_v, psum_hbm.at[pl.ds(off8, NH // 8)])

    half_pass(0, True)
    half_pass(NH, False)
    offc8 = pl.multiple_of(wid * (NPK // 8), 8)
    pltpu.sync_copy(cnt_v, pcnt_hbm.at[pl.ds(offc8, NPK // 8)])


def _sc_scatter(edge_out, col2d):
    eo2 = edge_out.reshape(E // 8, D)
    mesh = plsc.VectorSubcoreMesh(core_axis_name="c", subcore_axis_name="s")
    return pl.kernel(
        _sc_scatter_body,
        out_type=[
            jax.ShapeDtypeStruct((NW * NP // 8, D), jnp.float32),
            jax.ShapeDtypeStruct((NW * NPK // 8, D), jnp.float32),
        ],
        mesh=mesh,
        scratch_types=[
            pltpu.VMEM((SPB,), jnp.int32),
            pltpu.VMEM((SPB // 8, D), jnp.float32),
            pltpu.VMEM((NH // 8, D), jnp.float32),
            pltpu.VMEM((NPK // 8, D), jnp.float32),
        ],
    )(eo2, col2d)


def _cnt_body(pc_ref, mf_ref, out_ref):
    tot = jnp.sum(pc_ref[...], axis=0)
    out_ref[...] = jnp.dot(tot, mf_ref[...], preferred_element_type=jnp.float32)


def _cnt_unpack(pcnt, Mf):
    return pl.pallas_call(
        _cnt_body,
        grid=(1,),
        in_specs=[
            pl.BlockSpec((NW, NPK, 16), lambda i: (0, 0, 0)),
            pl.BlockSpec((16, 256), lambda i: (0, 0)),
        ],
        out_specs=pl.BlockSpec((NPK, 256), lambda i: (0, 0)),
        out_shape=jax.ShapeDtypeStruct((NPK, 256), jnp.float32),
    )(pcnt, Mf)


def _node_body(ps_ref, pc_ref, x_ref, w1_ref, b1_ref, w2_ref, b2_ref, out_ref):
    s = jnp.sum(ps_ref[...], axis=0)
    c = pc_ref[...]
    aggr = s / jnp.maximum(c, 1.0)
    xb = x_ref[...]
    w1a = w1_ref[:DE, :]
    w1b = w1_ref[DE:DE + D, :]
    g1 = _silu(jnp.dot(aggr, w1a, preferred_element_type=jnp.float32)
               + jnp.dot(xb, w1b, preferred_element_type=jnp.float32)
               + b1_ref[...])
    g2 = _silu(jnp.dot(g1, w2_ref[...], preferred_element_type=jnp.float32)
               + b2_ref[...])
    out_ref[...] = g2 + xb


def _node_mlp(psum, pcnt, x, Wn1, bn1, Wn2, bn2):
    blk = 1000
    grid = N // blk
    return pl.pallas_call(
        _node_body,
        grid=(grid,),
        in_specs=[
            pl.BlockSpec((NW, blk, DE), lambda i: (0, i, 0)),
            pl.BlockSpec((blk, DE), lambda i: (i, 0)),
            pl.BlockSpec((blk, D), lambda i: (i, 0)),
            pl.BlockSpec((DE + D, H), lambda i: (0, 0)),
            pl.BlockSpec((1, H), lambda i: (0, 0)),
            pl.BlockSpec((H, D), lambda i: (0, 0)),
            pl.BlockSpec((1, D), lambda i: (0, 0)),
        ],
        out_specs=pl.BlockSpec((blk, D), lambda i: (i, 0)),
        out_shape=jax.ShapeDtypeStruct((N, D), jnp.float32),
    )(psum, pcnt, x, Wn1, bn1.reshape(1, H), Wn2, bn2.reshape(1, D))


def kernel(x, edge_index, edge_attr, We1, be1, We2, be2, Wn1, bn1, Wn2, bn2):
    row2d = edge_index[0].reshape(TS, SPB)
    col2d = edge_index[1].reshape(TS, SPB)
    Xr, Xc = _project(x, We1, be1)
    G = _sc_gather(Xr, Xc, row2d, col2d)
    edge_out = _edge_mlp(G, edge_attr, We1, We2, be2)
    psum, pcnt = _sc_scatter(edge_out, col2d)
    psum = psum.reshape(NW, NP, DE)
    pcnt = pcnt.reshape(NW, NPK, 16)
    cnt_bc = _cnt_unpack(pcnt, jnp.asarray(_MF_NP)).reshape(NP, DE)
    x_out = _node_mlp(psum, cnt_bc, x, Wn1, bn1, Wn2, bn2)
    return (x_out, edge_out)


# speedup vs baseline: 2.7342x; 1.1217x over previous
import functools

import jax
import jax.numpy as jnp
from jax import lax
from jax.experimental import pallas as pl
from jax.experimental.pallas import tpu as pltpu
from jax.experimental.pallas import tpu_sc as plsc

N = 10000
E = 320000
D = 128
DE = 16
H = 128

NC = 2
NS = 16
NW = NC * NS

SPB = 128
TS = E // SPB
ST_BASE = TS // NW
ST_REM = TS % NW

TSP = 2560
WIN = 88

NP = 10240
ROWS_PER_TILE = NP // NS


import numpy as _np
_MF_NP = _np.repeat(_np.eye(16, dtype=_np.float32), 16, axis=1)


def _silu(v):
    return v * (1.0 / (1.0 + jnp.exp(-v)))


def _proj_body(x_ref, w_ref, b_ref, xr_ref, xc_ref):
    xb = x_ref[...]
    wr = w_ref[DE:DE + D, :]
    wc = w_ref[DE + D:DE + 2 * D, :]
    xr_ref[...] = jnp.dot(xb, wr, preferred_element_type=jnp.float32)
    xc_ref[...] = jnp.dot(xb, wc, preferred_element_type=jnp.float32) + b_ref[...]


def _project(x, We1, be1):
    blk = 1000
    grid = N // blk
    return pl.pallas_call(
        _proj_body,
        grid=(grid,),
        in_specs=[
            pl.BlockSpec((blk, D), lambda i: (i, 0)),
            pl.BlockSpec((DE + 2 * D, H), lambda i: (0, 0)),
            pl.BlockSpec((1, H), lambda i: (0, 0)),
        ],
        out_specs=[
            pl.BlockSpec((blk, H), lambda i: (i, 0)),
            pl.BlockSpec((blk, H), lambda i: (i, 0)),
        ],
        out_shape=[
            jax.ShapeDtypeStruct((N, H), jnp.float32),
            jax.ShapeDtypeStruct((N, H), jnp.float32),
        ],
    )(x, We1, be1.reshape(1, H))


def _sc_gather_body(xr_hbm, xc_hbm, row_hbm, col_hbm, g_hbm,
                    ridx_v, cidx_v, bufr_v, bufc_v, sem0, sem1):
    cid = lax.axis_index("c")
    sid = lax.axis_index("s")
    wid = sid * NC + cid
    start = wid * ST_BASE + jnp.minimum(wid, ST_REM)
    bstart = pl.multiple_of((start // 8) * 8, 8)
    pltpu.sync_copy(row_hbm.at[pl.ds(bstart, WIN)], ridx_v)
    pltpu.sync_copy(col_hbm.at[pl.ds(bstart, WIN)], cidx_v)

    def stream_body(t, carry):
        st = jnp.minimum(start + t, TS - 1)
        r = st - bstart
        a = pltpu.async_copy(xr_hbm.at[ridx_v.at[r]], bufr_v, sem0)
        b = pltpu.async_copy(xc_hbm.at[cidx_v.at[r]], bufc_v, sem1)
        a.wait()
        b.wait()

        @plsc.parallel_loop(0, SPB, unroll=2)
        def _add(i):
            for k in range(H // 16):
                sl = pl.ds(k * 16, 16)
                bufr_v[i, sl] = bufr_v[i, sl] + bufc_v[i, sl]

        pltpu.sync_copy(bufr_v, g_hbm.at[pl.ds(st * SPB, SPB)])
        return carry

    lax.fori_loop(0, ST_BASE + 1, stream_body, 0)


def _sc_gather(Xr, Xc, row2d, col2d):
    mesh = plsc.VectorSubcoreMesh(core_axis_name="c", subcore_axis_name="s")
    return pl.kernel(
        _sc_gather_body,
        out_type=jax.ShapeDtypeStruct((E, H), jnp.float32),
        mesh=mesh,
        scratch_types=[
            pltpu.VMEM((WIN, SPB), jnp.int32),
            pltpu.VMEM((WIN, SPB), jnp.int32),
            pltpu.VMEM((SPB, H), jnp.float32),
            pltpu.VMEM((SPB, H), jnp.float32),
            pltpu.SemaphoreType.DMA,
            pltpu.SemaphoreType.DMA,
        ],
    )(Xr, Xc, row2d, col2d)


def _edge_body(g_ref, ea_ref, wa_ref, w2_ref, b2_ref, out_ref):
    ea = ea_ref[...]
    h1 = _silu(g_ref[...] +
               jnp.dot(ea, wa_ref[...], preferred_element_type=jnp.float32))
    h2 = _silu(jnp.dot(h1, w2_ref[...], preferred_element_type=jnp.float32)
               + b2_ref[...])
    out_ref[...] = h2 + ea


def _edge_mlp(G, edge_attr, We1, We2, be2):
    blk = 1280
    grid = E // blk
    return pl.pallas_call(
        _edge_body,
        grid=(grid,),
        in_specs=[
            pl.BlockSpec((blk, H), lambda i: (i, 0)),
            pl.BlockSpec((blk, DE), lambda i: (i, 0)),
            pl.BlockSpec((DE, H), lambda i: (0, 0)),
            pl.BlockSpec((H, DE), lambda i: (0, 0)),
            pl.BlockSpec((1, DE), lambda i: (0, 0)),
        ],
        out_specs=pl.BlockSpec((blk, DE), lambda i: (i, 0)),
        out_shape=jax.ShapeDtypeStruct((E, DE), jnp.float32),
    )(G, edge_attr, We1[:DE, :], We2, be2.reshape(1, DE))


NH = NP // 2
NPK = NP // 16


def _sc_scatter_body(eo_hbm, col_hbm, psum_hbm, pcnt_hbm,
                     colv_v, val_v, acc_v, cnt_v):
    cid = lax.axis_index("c")
    sid = lax.axis_index("s")
    wid = sid * NC + cid
    n_st = ST_BASE + jnp.where(wid < ST_REM, 1, 0)
    start = wid * ST_BASE + jnp.minimum(wid, ST_REM)
    lane = lax.iota(jnp.int32, 16)
    bstart = pl.multiple_of((start // 8) * 8, 8)
    pltpu.sync_copy(col_hbm.at[pl.ds(bstart, WIN)], colv_v)

    @plsc.parallel_loop(0, NPK // 8, unroll=4)
    def _zc(i):
        for k in range(8):
            cnt_v[i, pl.ds(k * 16, 16)] = jnp.zeros((16,), jnp.float32)

    def half_pass(lo, with_counts):
        @plsc.parallel_loop(0, NH // 8, unroll=2)
        def _za(i):
            for k in range(8):
                acc_v[i, pl.ds(k * 16, 16)] = jnp.zeros((16,), jnp.float32)

        def stream_body(t, carry):
            st = start + t
            r = st - bstart
            pltpu.sync_copy(eo_hbm.at[pl.ds(st * (SPB // 8), SPB // 8)], val_v)

            def gbody(g, carry2):
                cv = colv_v[r, pl.ds(g * 16, 16)]
                for j in range(16):
                    c = cv[j]
                    rel = c - lo

                    @pl.when((rel >= 0) & (rel < NH))
                    def _(c=c, rel=rel, j=j):
                        row = rel // 8
                        off = (rel - row * 8) * 16
                        ev = val_v[g * 2 + j // 8, pl.ds((j % 8) * 16, 16)]
                        acc_v[row, pl.ds(off, 16)] = (
                            acc_v[row, pl.ds(off, 16)] + ev)

                    if with_counts:
                        rowc = c // 128
                        offc = ((c // 16) % 8) * 16
                        m = c % 16
                        cnt_v[rowc, pl.ds(offc, 16)] = (
                            cnt_v[rowc, pl.ds(offc, 16)]
                            + jnp.where(lane == m, 1.0, 0.0).astype(jnp.float32))
                return carry2

            lax.fori_loop(0, SPB // 16, gbody, 0)
            return carry

        lax.fori_loop(0, n_st, stream_body, 0)
        off8 = pl.multiple_of((wid * NP + lo) // 8, 8)
        pltpu.sync_copy(acc_v, psum_hbm.at[pl.ds(off8, NH // 8)])

    half_pass(0, True)
    half_pass(NH, False)
    offc8 = pl.multiple_of(wid * (NPK // 8), 8)
    pltpu.sync_copy(cnt_v, pcnt_hbm.at[pl.ds(offc8, NPK // 8)])


def _sc_scatter(edge_out, col2d):
    eo2 = edge_out.reshape(E // 8, D)
    mesh = plsc.VectorSubcoreMesh(core_axis_name="c", subcore_axis_name="s")
    return pl.kernel(
        _sc_scatter_body,
        out_type=[
            jax.ShapeDtypeStruct((NW * NP // 8, D), jnp.float32),
            jax.ShapeDtypeStruct((NW * NPK // 8, D), jnp.float32),
        ],
        mesh=mesh,
        scratch_types=[
            pltpu.VMEM((WIN, SPB), jnp.int32),
            pltpu.VMEM((SPB // 8, D), jnp.float32),
            pltpu.VMEM((NH // 8, D), jnp.float32),
            pltpu.VMEM((NPK // 8, D), jnp.float32),
        ],
    )(eo2, col2d)


def _cnt_body(pc_ref, mf_ref, out_ref):
    tot = jnp.sum(pc_ref[...], axis=0)
    out_ref[...] = jnp.dot(tot, mf_ref[...], preferred_element_type=jnp.float32)


def _cnt_unpack(pcnt, Mf):
    return pl.pallas_call(
        _cnt_body,
        grid=(1,),
        in_specs=[
            pl.BlockSpec((NW, NPK, 16), lambda i: (0, 0, 0)),
            pl.BlockSpec((16, 256), lambda i: (0, 0)),
        ],
        out_specs=pl.BlockSpec((NPK, 256), lambda i: (0, 0)),
        out_shape=jax.ShapeDtypeStruct((NPK, 256), jnp.float32),
    )(pcnt, Mf)


def _node_body(ps_ref, pc_ref, x_ref, w1_ref, b1_ref, w2_ref, b2_ref, out_ref):
    s = jnp.sum(ps_ref[...], axis=0)
    c = pc_ref[...]
    aggr = s / jnp.maximum(c, 1.0)
    xb = x_ref[...]
    w1a = w1_ref[:DE, :]
    w1b = w1_ref[DE:DE + D, :]
    g1 = _silu(jnp.dot(aggr, w1a, preferred_element_type=jnp.float32)
               + jnp.dot(xb, w1b, preferred_element_type=jnp.float32)
               + b1_ref[...])
    g2 = _silu(jnp.dot(g1, w2_ref[...], preferred_element_type=jnp.float32)
               + b2_ref[...])
    out_ref[...] = g2 + xb


def _node_mlp(psum, pcnt, x, Wn1, bn1, Wn2, bn2):
    blk = 1000
    grid = N // blk
    return pl.pallas_call(
        _node_body,
        grid=(grid,),
        in_specs=[
            pl.BlockSpec((NW, blk, DE), lambda i: (0, i, 0)),
            pl.BlockSpec((blk, DE), lambda i: (i, 0)),
            pl.BlockSpec((blk, D), lambda i: (i, 0)),
            pl.BlockSpec((DE + D, H), lambda i: (0, 0)),
            pl.BlockSpec((1, H), lambda i: (0, 0)),
            pl.BlockSpec((H, D), lambda i: (0, 0)),
            pl.BlockSpec((1, D), lambda i: (0, 0)),
        ],
        out_specs=pl.BlockSpec((blk, D), lambda i: (i, 0)),
        out_shape=jax.ShapeDtypeStruct((N, D), jnp.float32),
    )(psum, pcnt, x, Wn1, bn1.reshape(1, H), Wn2, bn2.reshape(1, D))


def kernel(x, edge_index, edge_attr, We1, be1, We2, be2, Wn1, bn1, Wn2, bn2):
    pad = jnp.zeros((TSP * SPB - E,), jnp.int32)
    row2d = jnp.concatenate([edge_index[0], pad]).reshape(TSP, SPB)
    col2d = jnp.concatenate([edge_index[1], pad]).reshape(TSP, SPB)
    Xr, Xc = _project(x, We1, be1)
    G = _sc_gather(Xr, Xc, row2d, col2d)
    edge_out = _edge_mlp(G, edge_attr, We1, We2, be2)
    psum, pcnt = _sc_scatter(edge_out, col2d)
    psum = psum.reshape(NW, NP, DE)
    pcnt = pcnt.reshape(NW, NPK, 16)
    cnt_bc = _cnt_unpack(pcnt, jnp.asarray(_MF_NP)).reshape(NP, DE)
    x_out = _node_mlp(psum, cnt_bc, x, Wn1, bn1, Wn2, bn2)
    return (x_out, edge_out)


# speedup vs baseline: 2.7402x; 1.0022x over previous
import functools

import jax
import jax.numpy as jnp
from jax import lax
from jax.experimental import pallas as pl
from jax.experimental.pallas import tpu as pltpu
from jax.experimental.pallas import tpu_sc as plsc

N = 10000
E = 320000
D = 128
DE = 16
H = 128

NC = 2
NS = 16
NW = NC * NS

SPB = 128
TS = E // SPB
ST_BASE = TS // NW
ST_REM = TS % NW

TSP = 2560
WIN = 88

NP = 10240
ROWS_PER_TILE = NP // NS


import numpy as _np
_MF_NP = _np.repeat(_np.eye(16, dtype=_np.float32), 16, axis=1)


def _silu(v):
    return v * (1.0 / (1.0 + jnp.exp(-v)))


def _proj_body(x_ref, w_ref, b_ref, xr_ref, xc_ref):
    xb = x_ref[...]
    wr = w_ref[DE:DE + D, :]
    wc = w_ref[DE + D:DE + 2 * D, :]
    xr_ref[...] = jnp.dot(xb, wr, preferred_element_type=jnp.float32)
    xc_ref[...] = jnp.dot(xb, wc, preferred_element_type=jnp.float32) + b_ref[...]


def _project(x, We1, be1):
    blk = 1000
    grid = N // blk
    return pl.pallas_call(
        _proj_body,
        grid=(grid,),
        in_specs=[
            pl.BlockSpec((blk, D), lambda i: (i, 0)),
            pl.BlockSpec((DE + 2 * D, H), lambda i: (0, 0)),
            pl.BlockSpec((1, H), lambda i: (0, 0)),
        ],
        out_specs=[
            pl.BlockSpec((blk, H), lambda i: (i, 0)),
            pl.BlockSpec((blk, H), lambda i: (i, 0)),
        ],
        out_shape=[
            jax.ShapeDtypeStruct((N, H), jnp.float32),
            jax.ShapeDtypeStruct((N, H), jnp.float32),
        ],
    )(x, We1, be1.reshape(1, H))


def _sc_gather_body(xr_hbm, xc_hbm, row_hbm, col_hbm, g_hbm,
                    ridx_v, cidx_v, bufr_v, bufc_v, sem0, sem1):
    cid = lax.axis_index("c")
    sid = lax.axis_index("s")
    wid = sid * NC + cid
    start = wid * ST_BASE + jnp.minimum(wid, ST_REM)
    bstart = pl.multiple_of((start // 8) * 8, 8)
    pltpu.sync_copy(row_hbm.at[pl.ds(bstart, WIN)], ridx_v)
    pltpu.sync_copy(col_hbm.at[pl.ds(bstart, WIN)], cidx_v)

    def stream_body(t, carry):
        st = jnp.minimum(start + t, TS - 1)
        r = st - bstart
        a = pltpu.async_copy(xr_hbm.at[ridx_v.at[r]], bufr_v, sem0)
        b = pltpu.async_copy(xc_hbm.at[cidx_v.at[r]], bufc_v, sem1)
        a.wait()
        b.wait()

        @plsc.parallel_loop(0, SPB, unroll=2)
        def _add(i):
            for k in range(H // 16):
                sl = pl.ds(k * 16, 16)
                bufr_v[i, sl] = bufr_v[i, sl] + bufc_v[i, sl]

        pltpu.sync_copy(bufr_v, g_hbm.at[pl.ds(st * SPB, SPB)])
        return carry

    lax.fori_loop(0, ST_BASE + 1, stream_body, 0)


def _sc_gather(Xr, Xc, row2d, col2d):
    mesh = plsc.VectorSubcoreMesh(core_axis_name="c", subcore_axis_name="s")
    return pl.kernel(
        _sc_gather_body,
        out_type=jax.ShapeDtypeStruct((E, H), jnp.float32),
        mesh=mesh,
        scratch_types=[
            pltpu.VMEM((WIN, SPB), jnp.int32),
            pltpu.VMEM((WIN, SPB), jnp.int32),
            pltpu.VMEM((SPB, H), jnp.float32),
            pltpu.VMEM((SPB, H), jnp.float32),
            pltpu.SemaphoreType.DMA,
            pltpu.SemaphoreType.DMA,
        ],
    )(Xr, Xc, row2d, col2d)


def _edge_body(g_ref, ea_ref, wa_ref, w2_ref, b2_ref, out_ref):
    ea = ea_ref[...]
    h1 = _silu(g_ref[...] +
               jnp.dot(ea, wa_ref[...], preferred_element_type=jnp.float32))
    h2 = _silu(jnp.dot(h1, w2_ref[...], preferred_element_type=jnp.float32)
               + b2_ref[...])
    out_ref[...] = h2 + ea


def _edge_mlp(G, edge_attr, We1, We2, be2):
    blk = 1280
    grid = E // blk
    return pl.pallas_call(
        _edge_body,
        grid=(grid,),
        in_specs=[
            pl.BlockSpec((blk, H), lambda i: (i, 0)),
            pl.BlockSpec((blk, DE), lambda i: (i, 0)),
            pl.BlockSpec((DE, H), lambda i: (0, 0)),
            pl.BlockSpec((H, DE), lambda i: (0, 0)),
            pl.BlockSpec((1, DE), lambda i: (0, 0)),
        ],
        out_specs=pl.BlockSpec((blk, DE), lambda i: (i, 0)),
        out_shape=jax.ShapeDtypeStruct((E, DE), jnp.float32),
    )(G, edge_attr, We1[:DE, :], We2, be2.reshape(1, DE))


NH = NP // 2
NPK = NP // 16


def _sc_scatter_body(eo_hbm, col_hbm, psum_hbm, pcnt_hbm,
                     colv_v, val_v, acc_v, cnt_v):
    cid = lax.axis_index("c")
    sid = lax.axis_index("s")
    wid = sid * NC + cid
    n_st = ST_BASE + jnp.where(wid < ST_REM, 1, 0)
    start = wid * ST_BASE + jnp.minimum(wid, ST_REM)
    lane = lax.iota(jnp.int32, 16)
    bstart = pl.multiple_of((start // 8) * 8, 8)
    pltpu.sync_copy(col_hbm.at[pl.ds(bstart, WIN)], colv_v)

    @plsc.parallel_loop(0, NPK // 8, unroll=4)
    def _zc(i):
        for k in range(8):
            cnt_v[i, pl.ds(k * 16, 16)] = jnp.zeros((16,), jnp.float32)

    def half_pass(lo, with_counts):
        @plsc.parallel_loop(0, NH // 8, unroll=2)
        def _za(i):
            for k in range(8):
                acc_v[i, pl.ds(k * 16, 16)] = jnp.zeros((16,), jnp.float32)

        def stream_body(t, carry):
            st = start + t
            r = st - bstart
            pltpu.sync_copy(eo_hbm.at[pl.ds(st * SPB, SPB)], val_v)

            def gbody(g, carry2):
                cv = colv_v[r, pl.ds(g * 16, 16)]
                for j in range(16):
                    c = cv[j]
                    rel = c - lo

                    @pl.when((rel >= 0) & (rel < NH))
                    def _(c=c, rel=rel, j=j):
                        row = rel // 8
                        off = (rel - row * 8) * 16
                        ev = val_v[g * 16 + j, :]
                        acc_v[row, pl.ds(off, 16)] = (
                            acc_v[row, pl.ds(off, 16)] + ev)

                    if with_counts:
                        rowc = c // 128
                        offc = ((c // 16) % 8) * 16
                        m = c % 16
                        cnt_v[rowc, pl.ds(offc, 16)] = (
                            cnt_v[rowc, pl.ds(offc, 16)]
                            + jnp.where(lane == m, 1.0, 0.0).astype(jnp.float32))
                return carry2

            lax.fori_loop(0, SPB // 16, gbody, 0)
            return carry

        lax.fori_loop(0, n_st, stream_body, 0)
        off8 = pl.multiple_of((wid * NP + lo) // 8, 8)
        pltpu.sync_copy(acc_v, psum_hbm.at[pl.ds(off8, NH // 8)])

    half_pass(0, True)
    half_pass(NH, False)
    offc8 = pl.multiple_of(wid * (NPK // 8), 8)
    pltpu.sync_copy(cnt_v, pcnt_hbm.at[pl.ds(offc8, NPK // 8)])


def _sc_scatter(edge_out, col2d):
    mesh = plsc.VectorSubcoreMesh(core_axis_name="c", subcore_axis_name="s")
    return pl.kernel(
        _sc_scatter_body,
        out_type=[
            jax.ShapeDtypeStruct((NW * NP // 8, D), jnp.float32),
            jax.ShapeDtypeStruct((NW * NPK // 8, D), jnp.float32),
        ],
        mesh=mesh,
        scratch_types=[
            pltpu.VMEM((WIN, SPB), jnp.int32),
            pltpu.VMEM((SPB, DE), jnp.float32),
            pltpu.VMEM((NH // 8, D), jnp.float32),
            pltpu.VMEM((NPK // 8, D), jnp.float32),
        ],
    )(edge_out, col2d)


def _cnt_body(pc_ref, mf_ref, out_ref):
    tot = jnp.sum(pc_ref[...], axis=0)
    out_ref[...] = jnp.dot(tot, mf_ref[...], preferred_element_type=jnp.float32)


def _cnt_unpack(pcnt, Mf):
    return pl.pallas_call(
        _cnt_body,
        grid=(1,),
        in_specs=[
            pl.BlockSpec((NW, NPK, 16), lambda i: (0, 0, 0)),
            pl.BlockSpec((16, 256), lambda i: (0, 0)),
        ],
        out_specs=pl.BlockSpec((NPK, 256), lambda i: (0, 0)),
        out_shape=jax.ShapeDtypeStruct((NPK, 256), jnp.float32),
    )(pcnt, Mf)


def _node_body(ps_ref, pc_ref, x_ref, w1_ref, b1_ref, w2_ref, b2_ref, out_ref):
    s = jnp.sum(ps_ref[...], axis=0)
    c = pc_ref[...]
    aggr = s / jnp.maximum(c, 1.0)
    xb = x_ref[...]
    w1a = w1_ref[:DE, :]
    w1b = w1_ref[DE:DE + D, :]
    g1 = _silu(jnp.dot(aggr, w1a, preferred_element_type=jnp.float32)
               + jnp.dot(xb, w1b, preferred_element_type=jnp.float32)
               + b1_ref[...])
    g2 = _silu(jnp.dot(g1, w2_ref[...], preferred_element_type=jnp.float32)
               + b2_ref[...])
    out_ref[...] = g2 + xb


def _node_mlp(psum, pcnt, x, Wn1, bn1, Wn2, bn2):
    blk = 1000
    grid = N // blk
    return pl.pallas_call(
        _node_body,
        grid=(grid,),
        in_specs=[
            pl.BlockSpec((NW, blk, DE), lambda i: (0, i, 0)),
            pl.BlockSpec((blk, DE), lambda i: (i, 0)),
            pl.BlockSpec((blk, D), lambda i: (i, 0)),
            pl.BlockSpec((DE + D, H), lambda i: (0, 0)),
            pl.BlockSpec((1, H), lambda i: (0, 0)),
            pl.BlockSpec((H, D), lambda i: (0, 0)),
            pl.BlockSpec((1, D), lambda i: (0, 0)),
        ],
        out_specs=pl.BlockSpec((blk, D), lambda i: (i, 0)),
        out_shape=jax.ShapeDtypeStruct((N, D), jnp.float32),
    )(psum, pcnt, x, Wn1, bn1.reshape(1, H), Wn2, bn2.reshape(1, D))


def kernel(x, edge_index, edge_attr, We1, be1, We2, be2, Wn1, bn1, Wn2, bn2):
    pad = jnp.zeros((TSP * SPB - E,), jnp.int32)
    row2d = jnp.concatenate([edge_index[0], pad]).reshape(TSP, SPB)
    col2d = jnp.concatenate([edge_index[1], pad]).reshape(TSP, SPB)
    Xr, Xc = _project(x, We1, be1)
    G = _sc_gather(Xr, Xc, row2d, col2d)
    edge_out = _edge_mlp(G, edge_attr, We1, We2, be2)
    psum, pcnt = _sc_scatter(edge_out, col2d)
    psum = psum.reshape(NW, NP, DE)
    pcnt = pcnt.reshape(NW, NPK, 16)
    cnt_bc = _cnt_unpack(pcnt, jnp.asarray(_MF_NP)).reshape(NP, DE)
    x_out = _node_mlp(psum, cnt_bc, x, Wn1, bn1, Wn2, bn2)
    return (x_out, edge_out)


# speedup vs baseline: 2.9691x; 1.0835x over previous
import functools

import jax
import jax.numpy as jnp
from jax import lax
from jax.experimental import pallas as pl
from jax.experimental.pallas import tpu as pltpu
from jax.experimental.pallas import tpu_sc as plsc

N = 10000
E = 320000
D = 128
DE = 16
H = 128

NC = 2
NS = 16
NW = NC * NS

SPB = 128
TS = E // SPB
ST_BASE = TS // NW
ST_REM = TS % NW

TSP = 2560
WIN = 88

NP = 10240
ROWS_PER_TILE = NP // NS


import numpy as _np
_MF_NP = _np.repeat(_np.eye(16, dtype=_np.float32), 16, axis=1)


def _silu(v):
    return v * (1.0 / (1.0 + jnp.exp(-v)))


def _proj_body(x_ref, w_ref, b_ref, xr_ref, xc_ref):
    xb = x_ref[...]
    wr = w_ref[DE:DE + D, :]
    wc = w_ref[DE + D:DE + 2 * D, :]
    xr_ref[...] = jnp.dot(xb, wr, preferred_element_type=jnp.float32)
    xc_ref[...] = jnp.dot(xb, wc, preferred_element_type=jnp.float32) + b_ref[...]


def _project(x, We1, be1):
    blk = 1000
    grid = N // blk
    return pl.pallas_call(
        _proj_body,
        grid=(grid,),
        in_specs=[
            pl.BlockSpec((blk, D), lambda i: (i, 0)),
            pl.BlockSpec((DE + 2 * D, H), lambda i: (0, 0)),
            pl.BlockSpec((1, H), lambda i: (0, 0)),
        ],
        out_specs=[
            pl.BlockSpec((blk, H), lambda i: (i, 0)),
            pl.BlockSpec((blk, H), lambda i: (i, 0)),
        ],
        out_shape=[
            jax.ShapeDtypeStruct((N, H), jnp.float32),
            jax.ShapeDtypeStruct((N, H), jnp.float32),
        ],
    )(x, We1, be1.reshape(1, H))


def _sc_gather_body(xr_hbm, xc_hbm, row_hbm, col_hbm, g_hbm,
                    ridx_v, cidx_v, bufr_v, bufc_v, sem0, sem1):
    cid = lax.axis_index("c")
    sid = lax.axis_index("s")
    wid = sid * NC + cid
    start = wid * ST_BASE + jnp.minimum(wid, ST_REM)
    bstart = pl.multiple_of((start // 8) * 8, 8)
    pltpu.sync_copy(row_hbm.at[pl.ds(bstart, WIN)], ridx_v)
    pltpu.sync_copy(col_hbm.at[pl.ds(bstart, WIN)], cidx_v)

    def stream_body(t, carry):
        st = jnp.minimum(start + t, TS - 1)
        r = st - bstart
        a = pltpu.async_copy(xr_hbm.at[ridx_v.at[r]], bufr_v, sem0)
        b = pltpu.async_copy(xc_hbm.at[cidx_v.at[r]], bufc_v, sem1)
        a.wait()
        b.wait()

        @plsc.parallel_loop(0, SPB, unroll=2)
        def _add(i):
            for k in range(H // 16):
                sl = pl.ds(k * 16, 16)
                bufr_v[i, sl] = bufr_v[i, sl] + bufc_v[i, sl]

        pltpu.sync_copy(bufr_v, g_hbm.at[pl.ds(st * SPB, SPB)])
        return carry

    lax.fori_loop(0, ST_BASE + 1, stream_body, 0)


def _sc_gather(Xr, Xc, row2d, col2d):
    mesh = plsc.VectorSubcoreMesh(core_axis_name="c", subcore_axis_name="s")
    return pl.kernel(
        _sc_gather_body,
        out_type=jax.ShapeDtypeStruct((E, H), jnp.float32),
        mesh=mesh,
        scratch_types=[
            pltpu.VMEM((WIN, SPB), jnp.int32),
            pltpu.VMEM((WIN, SPB), jnp.int32),
            pltpu.VMEM((SPB, H), jnp.float32),
            pltpu.VMEM((SPB, H), jnp.float32),
            pltpu.SemaphoreType.DMA,
            pltpu.SemaphoreType.DMA,
        ],
    )(Xr, Xc, row2d, col2d)


def _edge_body(g_ref, ea_ref, wa_ref, w2_ref, b2_ref, out_ref):
    ea = ea_ref[...]
    h1 = _silu(g_ref[...] +
               jnp.dot(ea, wa_ref[...], preferred_element_type=jnp.float32))
    h2 = _silu(jnp.dot(h1, w2_ref[...], preferred_element_type=jnp.float32)
               + b2_ref[...])
    out_ref[...] = h2 + ea


def _edge_mlp(G, edge_attr, We1, We2, be2):
    blk = 3200
    grid = E // blk
    return pl.pallas_call(
        _edge_body,
        grid=(grid,),
        in_specs=[
            pl.BlockSpec((blk, H), lambda i: (i, 0)),
            pl.BlockSpec((blk, DE), lambda i: (i, 0)),
            pl.BlockSpec((DE, H), lambda i: (0, 0)),
            pl.BlockSpec((H, DE), lambda i: (0, 0)),
            pl.BlockSpec((1, DE), lambda i: (0, 0)),
        ],
        out_specs=pl.BlockSpec((blk, DE), lambda i: (i, 0)),
        out_shape=jax.ShapeDtypeStruct((E, DE), jnp.float32),
    )(G, edge_attr, We1[:DE, :], We2, be2.reshape(1, DE))


NH = NP // 2
NPK = NP // 16


def _sc_scatter_body(eo_hbm, col_hbm, psum_hbm, pcnt_hbm,
                     colv_v, val_v, acc_v, cnt_v):
    cid = lax.axis_index("c")
    sid = lax.axis_index("s")
    wid = sid * NC + cid
    n_st = ST_BASE + jnp.where(wid < ST_REM, 1, 0)
    start = wid * ST_BASE + jnp.minimum(wid, ST_REM)
    lane = lax.iota(jnp.int32, 16)
    bstart = pl.multiple_of((start // 8) * 8, 8)
    pltpu.sync_copy(col_hbm.at[pl.ds(bstart, WIN)], colv_v)

    @plsc.parallel_loop(0, NPK // 8, unroll=4)
    def _zc(i):
        for k in range(8):
            cnt_v[i, pl.ds(k * 16, 16)] = jnp.zeros((16,), jnp.float32)

    def half_pass(lo, with_counts):
        @plsc.parallel_loop(0, NH // 8, unroll=2)
        def _za(i):
            for k in range(8):
                acc_v[i, pl.ds(k * 16, 16)] = jnp.zeros((16,), jnp.float32)

        def stream_body(t, carry):
            st = start + t
            r = st - bstart
            pltpu.sync_copy(eo_hbm.at[pl.ds(st * SPB, SPB)], val_v)

            def gbody(g, carry2):
                cv = colv_v[r, pl.ds(g * 16, 16)]
                for j in range(16):
                    c = cv[j]
                    rel = c - lo

                    @pl.when((rel >= 0) & (rel < NH))
                    def _(c=c, rel=rel, j=j):
                        row = rel // 8
                        off = (rel - row * 8) * 16
                        ev = val_v[g * 16 + j, :]
                        acc_v[row, pl.ds(off, 16)] = (
                            acc_v[row, pl.ds(off, 16)] + ev)

                    if with_counts:
                        rowc = c // 128
                        offc = ((c // 16) % 8) * 16
                        m = c % 16
                        cnt_v[rowc, pl.ds(offc, 16)] = (
                            cnt_v[rowc, pl.ds(offc, 16)]
                            + jnp.where(lane == m, 1.0, 0.0).astype(jnp.float32))
                return carry2

            lax.fori_loop(0, SPB // 16, gbody, 0)
            return carry

        lax.fori_loop(0, n_st, stream_body, 0)
        off8 = pl.multiple_of((wid * NP + lo) // 8, 8)
        pltpu.sync_copy(acc_v, psum_hbm.at[pl.ds(off8, NH // 8)])

    half_pass(0, True)
    half_pass(NH, False)
    offc8 = pl.multiple_of(wid * (NPK // 8), 8)
    pltpu.sync_copy(cnt_v, pcnt_hbm.at[pl.ds(offc8, NPK // 8)])


def _sc_scatter(edge_out, col2d):
    mesh = plsc.VectorSubcoreMesh(core_axis_name="c", subcore_axis_name="s")
    return pl.kernel(
        _sc_scatter_body,
        out_type=[
            jax.ShapeDtypeStruct((NW * NP // 8, D), jnp.float32),
            jax.ShapeDtypeStruct((NW * NPK // 8, D), jnp.float32),
        ],
        mesh=mesh,
        scratch_types=[
            pltpu.VMEM((WIN, SPB), jnp.int32),
            pltpu.VMEM((SPB, DE), jnp.float32),
            pltpu.VMEM((NH // 8, D), jnp.float32),
            pltpu.VMEM((NPK // 8, D), jnp.float32),
        ],
    )(edge_out, col2d)


def _cnt_body(pc_ref, mf_ref, out_ref):
    tot = jnp.sum(pc_ref[...], axis=0)
    out_ref[...] = jnp.dot(tot, mf_ref[...], preferred_element_type=jnp.float32)


def _cnt_unpack(pcnt, Mf):
    return pl.pallas_call(
        _cnt_body,
        grid=(1,),
        in_specs=[
            pl.BlockSpec((NW, NPK, 16), lambda i: (0, 0, 0)),
            pl.BlockSpec((16, 256), lambda i: (0, 0)),
        ],
        out_specs=pl.BlockSpec((NPK, 256), lambda i: (0, 0)),
        out_shape=jax.ShapeDtypeStruct((NPK, 256), jnp.float32),
    )(pcnt, Mf)


def _node_body(ps_ref, pc_ref, x_ref, w1_ref, b1_ref, w2_ref, b2_ref, out_ref):
    s = jnp.sum(ps_ref[...], axis=0)
    c = pc_ref[...]
    aggr = s / jnp.maximum(c, 1.0)
    xb = x_ref[...]
    w1a = w1_ref[:DE, :]
    w1b = w1_ref[DE:DE + D, :]
    g1 = _silu(jnp.dot(aggr, w1a, preferred_element_type=jnp.float32)
               + jnp.dot(xb, w1b, preferred_element_type=jnp.float32)
               + b1_ref[...])
    g2 = _silu(jnp.dot(g1, w2_ref[...], preferred_element_type=jnp.float32)
               + b2_ref[...])
    out_ref[...] = g2 + xb


def _node_mlp(psum, pcnt, x, Wn1, bn1, Wn2, bn2):
    blk = 1000
    grid = N // blk
    return pl.pallas_call(
        _node_body,
        grid=(grid,),
        in_specs=[
            pl.BlockSpec((NW, blk, DE), lambda i: (0, i, 0)),
            pl.BlockSpec((blk, DE), lambda i: (i, 0)),
            pl.BlockSpec((blk, D), lambda i: (i, 0)),
            pl.BlockSpec((DE + D, H), lambda i: (0, 0)),
            pl.BlockSpec((1, H), lambda i: (0, 0)),
            pl.BlockSpec((H, D), lambda i: (0, 0)),
            pl.BlockSpec((1, D), lambda i: (0, 0)),
        ],
        out_specs=pl.BlockSpec((blk, D), lambda i: (i, 0)),
        out_shape=jax.ShapeDtypeStruct((N, D), jnp.float32),
    )(psum, pcnt, x, Wn1, bn1.reshape(1, H), Wn2, bn2.reshape(1, D))


def kernel(x, edge_index, edge_attr, We1, be1, We2, be2, Wn1, bn1, Wn2, bn2):
    pad = jnp.zeros((TSP * SPB - E,), jnp.int32)
    row2d = jnp.concatenate([edge_index[0], pad]).reshape(TSP, SPB)
    col2d = jnp.concatenate([edge_index[1], pad]).reshape(TSP, SPB)
    Xr, Xc = _project(x, We1, be1)
    G = _sc_gather(Xr, Xc, row2d, col2d)
    edge_out = _edge_mlp(G, edge_attr, We1, We2, be2)
    psum, pcnt = _sc_scatter(edge_out, col2d)
    psum = psum.reshape(NW, NP, DE)
    pcnt = pcnt.reshape(NW, NPK, 16)
    cnt_bc = _cnt_unpack(pcnt, jnp.asarray(_MF_NP)).reshape(NP, DE)
    x_out = _node_mlp(psum, cnt_bc, x, Wn1, bn1, Wn2, bn2)
    return (x_out, edge_out)


# speedup vs baseline: 3.0498x; 1.0272x over previous
import functools

import jax
import jax.numpy as jnp
from jax import lax
from jax.experimental import pallas as pl
from jax.experimental.pallas import tpu as pltpu
from jax.experimental.pallas import tpu_sc as plsc

N = 10000
E = 320000
D = 128
DE = 16
H = 128

NC = 2
NS = 16
NW = NC * NS

SPB = 128
TS = E // SPB
ST_BASE = TS // NW
ST_REM = TS % NW

TSP = 2560
WIN = 88

NP = 10240
ROWS_PER_TILE = NP // NS


import numpy as _np
_MF_NP = _np.repeat(_np.eye(16, dtype=_np.float32), 16, axis=1)


def _silu(v):
    return v * (1.0 / (1.0 + jnp.exp(-v)))


def _proj_body(x_ref, w_ref, b_ref, xr_ref, xc_ref):
    xb = x_ref[...]
    wr = w_ref[DE:DE + D, :]
    wc = w_ref[DE + D:DE + 2 * D, :]
    xr_ref[...] = jnp.dot(xb, wr, preferred_element_type=jnp.float32)
    xc_ref[...] = jnp.dot(xb, wc, preferred_element_type=jnp.float32) + b_ref[...]


def _project(x, We1, be1):
    blk = 2000
    grid = N // blk
    return pl.pallas_call(
        _proj_body,
        grid=(grid,),
        in_specs=[
            pl.BlockSpec((blk, D), lambda i: (i, 0)),
            pl.BlockSpec((DE + 2 * D, H), lambda i: (0, 0)),
            pl.BlockSpec((1, H), lambda i: (0, 0)),
        ],
        out_specs=[
            pl.BlockSpec((blk, H), lambda i: (i, 0)),
            pl.BlockSpec((blk, H), lambda i: (i, 0)),
        ],
        out_shape=[
            jax.ShapeDtypeStruct((N, H), jnp.float32),
            jax.ShapeDtypeStruct((N, H), jnp.float32),
        ],
    )(x, We1, be1.reshape(1, H))


def _sc_gather_body(xr_hbm, xc_hbm, row_hbm, col_hbm, g_hbm,
                    ridx_v, cidx_v, bufr_v, bufc_v, sem0, sem1):
    cid = lax.axis_index("c")
    sid = lax.axis_index("s")
    wid = sid * NC + cid
    start = wid * ST_BASE + jnp.minimum(wid, ST_REM)
    bstart = pl.multiple_of((start // 8) * 8, 8)
    pltpu.sync_copy(row_hbm.at[pl.ds(bstart, WIN)], ridx_v)
    pltpu.sync_copy(col_hbm.at[pl.ds(bstart, WIN)], cidx_v)

    def stream_body(t, carry):
        st = jnp.minimum(start + t, TS - 1)
        r = st - bstart
        a = pltpu.async_copy(xr_hbm.at[ridx_v.at[r]], bufr_v, sem0)
        b = pltpu.async_copy(xc_hbm.at[cidx_v.at[r]], bufc_v, sem1)
        a.wait()
        b.wait()

        @plsc.parallel_loop(0, SPB, unroll=2)
        def _add(i):
            for k in range(H // 16):
                sl = pl.ds(k * 16, 16)
                bufr_v[i, sl] = bufr_v[i, sl] + bufc_v[i, sl]

        pltpu.sync_copy(bufr_v, g_hbm.at[pl.ds(st * SPB, SPB)])
        return carry

    lax.fori_loop(0, ST_BASE + 1, stream_body, 0)


def _sc_gather(Xr, Xc, row2d, col2d):
    mesh = plsc.VectorSubcoreMesh(core_axis_name="c", subcore_axis_name="s")
    return pl.kernel(
        _sc_gather_body,
        out_type=jax.ShapeDtypeStruct((E, H), jnp.float32),
        mesh=mesh,
        scratch_types=[
            pltpu.VMEM((WIN, SPB), jnp.int32),
            pltpu.VMEM((WIN, SPB), jnp.int32),
            pltpu.VMEM((SPB, H), jnp.float32),
            pltpu.VMEM((SPB, H), jnp.float32),
            pltpu.SemaphoreType.DMA,
            pltpu.SemaphoreType.DMA,
        ],
    )(Xr, Xc, row2d, col2d)


def _edge_body(g_ref, ea_ref, wa_ref, w2_ref, b2_ref, out_ref):
    ea = ea_ref[...]
    h1 = _silu(g_ref[...] +
               jnp.dot(ea, wa_ref[...], preferred_element_type=jnp.float32))
    h2 = _silu(jnp.dot(h1, w2_ref[...], preferred_element_type=jnp.float32)
               + b2_ref[...])
    out_ref[...] = h2 + ea


def _edge_mlp(G, edge_attr, We1, We2, be2):
    blk = 6400
    grid = E // blk
    return pl.pallas_call(
        _edge_body,
        grid=(grid,),
        in_specs=[
            pl.BlockSpec((blk, H), lambda i: (i, 0)),
            pl.BlockSpec((blk, DE), lambda i: (i, 0)),
            pl.BlockSpec((DE, H), lambda i: (0, 0)),
            pl.BlockSpec((H, DE), lambda i: (0, 0)),
            pl.BlockSpec((1, DE), lambda i: (0, 0)),
        ],
        out_specs=pl.BlockSpec((blk, DE), lambda i: (i, 0)),
        out_shape=jax.ShapeDtypeStruct((E, DE), jnp.float32),
    )(G, edge_attr, We1[:DE, :], We2, be2.reshape(1, DE))


NH = NP // 2
NPK = NP // 16


def _sc_scatter_body(eo_hbm, col_hbm, psum_hbm, pcnt_hbm,
                     colv_v, val_v, acc_v, cnt_v):
    cid = lax.axis_index("c")
    sid = lax.axis_index("s")
    wid = sid * NC + cid
    n_st = ST_BASE + jnp.where(wid < ST_REM, 1, 0)
    start = wid * ST_BASE + jnp.minimum(wid, ST_REM)
    lane = lax.iota(jnp.int32, 16)
    bstart = pl.multiple_of((start // 8) * 8, 8)
    pltpu.sync_copy(col_hbm.at[pl.ds(bstart, WIN)], colv_v)

    @plsc.parallel_loop(0, NPK // 8, unroll=4)
    def _zc(i):
        for k in range(8):
            cnt_v[i, pl.ds(k * 16, 16)] = jnp.zeros((16,), jnp.float32)

    def half_pass(lo, with_counts):
        @plsc.parallel_loop(0, NH // 8, unroll=2)
        def _za(i):
            for k in range(8):
                acc_v[i, pl.ds(k * 16, 16)] = jnp.zeros((16,), jnp.float32)

        def stream_body(t, carry):
            st = start + t
            r = st - bstart
            pltpu.sync_copy(eo_hbm.at[pl.ds(st * SPB, SPB)], val_v)

            def gbody(g, carry2):
                cv = colv_v[r, pl.ds(g * 16, 16)]
                for j in range(16):
                    c = cv[j]
                    rel = c - lo

                    @pl.when((rel >= 0) & (rel < NH))
                    def _(c=c, rel=rel, j=j):
                        row = rel // 8
                        off = (rel - row * 8) * 16
                        ev = val_v[g * 16 + j, :]
                        acc_v[row, pl.ds(off, 16)] = (
                            acc_v[row, pl.ds(off, 16)] + ev)

                    if with_counts:
                        rowc = c // 128
                        offc = ((c // 16) % 8) * 16
                        m = c % 16
                        cnt_v[rowc, pl.ds(offc, 16)] = (
                            cnt_v[rowc, pl.ds(offc, 16)]
                            + jnp.where(lane == m, 1.0, 0.0).astype(jnp.float32))
                return carry2

            lax.fori_loop(0, SPB // 16, gbody, 0)
            return carry

        lax.fori_loop(0, n_st, stream_body, 0)
        off8 = pl.multiple_of((wid * NP + lo) // 8, 8)
        pltpu.sync_copy(acc_v, psum_hbm.at[pl.ds(off8, NH // 8)])

    half_pass(0, True)
    half_pass(NH, False)
    offc8 = pl.multiple_of(wid * (NPK // 8), 8)
    pltpu.sync_copy(cnt_v, pcnt_hbm.at[pl.ds(offc8, NPK // 8)])


def _sc_scatter(edge_out, col2d):
    mesh = plsc.VectorSubcoreMesh(core_axis_name="c", subcore_axis_name="s")
    return pl.kernel(
        _sc_scatter_body,
        out_type=[
            jax.ShapeDtypeStruct((NW * NP // 8, D), jnp.float32),
            jax.ShapeDtypeStruct((NW * NPK // 8, D), jnp.float32),
        ],
        mesh=mesh,
        scratch_types=[
            pltpu.VMEM((WIN, SPB), jnp.int32),
            pltpu.VMEM((SPB, DE), jnp.float32),
            pltpu.VMEM((NH // 8, D), jnp.float32),
            pltpu.VMEM((NPK // 8, D), jnp.float32),
        ],
    )(edge_out, col2d)


def _cnt_body(pc_ref, mf_ref, out_ref):
    tot = jnp.sum(pc_ref[...], axis=0)
    out_ref[...] = jnp.dot(tot, mf_ref[...], preferred_element_type=jnp.float32)


def _cnt_unpack(pcnt, Mf):
    return pl.pallas_call(
        _cnt_body,
        grid=(1,),
        in_specs=[
            pl.BlockSpec((NW, NPK, 16), lambda i: (0, 0, 0)),
            pl.BlockSpec((16, 256), lambda i: (0, 0)),
        ],
        out_specs=pl.BlockSpec((NPK, 256), lambda i: (0, 0)),
        out_shape=jax.ShapeDtypeStruct((NPK, 256), jnp.float32),
    )(pcnt, Mf)


def _node_body(ps_ref, pc_ref, x_ref, w1_ref, b1_ref, w2_ref, b2_ref, out_ref):
    s = jnp.sum(ps_ref[...], axis=0)
    c = pc_ref[...]
    aggr = s / jnp.maximum(c, 1.0)
    xb = x_ref[...]
    w1a = w1_ref[:DE, :]
    w1b = w1_ref[DE:DE + D, :]
    g1 = _silu(jnp.dot(aggr, w1a, preferred_element_type=jnp.float32)
               + jnp.dot(xb, w1b, preferred_element_type=jnp.float32)
               + b1_ref[...])
    g2 = _silu(jnp.dot(g1, w2_ref[...], preferred_element_type=jnp.float32)
               + b2_ref[...])
    out_ref[...] = g2 + xb


def _node_mlp(psum, pcnt, x, Wn1, bn1, Wn2, bn2):
    blk = 1000
    grid = N // blk
    return pl.pallas_call(
        _node_body,
        grid=(grid,),
        in_specs=[
            pl.BlockSpec((NW, blk, DE), lambda i: (0, i, 0)),
            pl.BlockSpec((blk, DE), lambda i: (i, 0)),
            pl.BlockSpec((blk, D), lambda i: (i, 0)),
            pl.BlockSpec((DE + D, H), lambda i: (0, 0)),
            pl.BlockSpec((1, H), lambda i: (0, 0)),
            pl.BlockSpec((H, D), lambda i: (0, 0)),
            pl.BlockSpec((1, D), lambda i: (0, 0)),
        ],
        out_specs=pl.BlockSpec((blk, D), lambda i: (i, 0)),
        out_shape=jax.ShapeDtypeStruct((N, D), jnp.float32),
    )(psum, pcnt, x, Wn1, bn1.reshape(1, H), Wn2, bn2.reshape(1, D))


def kernel(x, edge_index, edge_attr, We1, be1, We2, be2, Wn1, bn1, Wn2, bn2):
    pad = jnp.zeros((TSP * SPB - E,), jnp.int32)
    row2d = jnp.concatenate([edge_index[0], pad]).reshape(TSP, SPB)
    col2d = jnp.concatenate([edge_index[1], pad]).reshape(TSP, SPB)
    Xr, Xc = _project(x, We1, be1)
    G = _sc_gather(Xr, Xc, row2d, col2d)
    edge_out = _edge_mlp(G, edge_attr, We1, We2, be2)
    psum, pcnt = _sc_scatter(edge_out, col2d)
    psum = psum.reshape(NW, NP, DE)
    pcnt = pcnt.reshape(NW, NPK, 16)
    cnt_bc = _cnt_unpack(pcnt, jnp.asarray(_MF_NP)).reshape(NP, DE)
    x_out = _node_mlp(psum, cnt_bc, x, Wn1, bn1, Wn2, bn2)
    return (x_out, edge_out)
